# baseline (speedup 1.0000x reference)
import functools

import jax
import jax.numpy as jnp
import numpy as np
from jax.experimental import pallas as pl

_B = 8
_N = 512
_D = 128
_TEMPERATURE = 0.07
_ALPHA = 0.5
_TOP_K = 5
_SIGMA = 2.0
_LOSS_W = 1.0
_EPS = 1e-09

_CHUNK = 128
_BANDW = 256
_BANDC = [0, 64, 192, 256]

_ii = np.arange(_N)
_startc = np.clip(_ii - 2, 0, _N - _TOP_K)[:, None]
_cc = _ii[None, :]
_wmask = (_cc >= _startc) & (_cc < _startc + _TOP_K) & (_cc != _ii[:, None])
_dd = (_ii[:, None] - _cc).astype(np.float32)
_posw = np.exp(-(_dd * _dd) / np.float32(2.0 * _SIGMA * _SIGMA))
_C1 = (_wmask * (_ALPHA * _posw)).astype(np.float32)
_C2 = (_wmask * ((1.0 - _ALPHA) / 3.0)).astype(np.float32)


def _softmax_noshift(s):
    e = jnp.exp(s)
    return e * (1.0 / jnp.sum(e, axis=-1, keepdims=True))


def _dot_bf16(x, y):
    return jnp.dot(x.astype(jnp.bfloat16), y.astype(jnp.bfloat16),
                   preferred_element_type=jnp.float32)


def _band_num(a, e, c1, c2):
    t = (c1 + c2 * a) * e
    return jnp.sum(t, axis=-1, keepdims=True)


def _loss_kernel(z1_ref, z2_ref, out_ref):
    z1 = z1_ref[0]
    z2 = z2_ref[0]
    n = _N
    col = jax.lax.broadcasted_iota(jnp.int32, (n, n), 1)
    row = jax.lax.broadcasted_iota(jnp.int32, (n, n), 0)
    start = jnp.clip(row - 2, 0, n - _TOP_K)
    w = (col >= start) & (col < start + _TOP_K) & (col != row)
    d = (row - col).astype(jnp.float32)
    pos_w = jnp.exp(-(d * d) * (1.0 / (2.0 * _SIGMA * _SIGMA)))
    c1 = jnp.where(w, _ALPHA * pos_w, 0.0)
    c2 = jnp.where(w, (1.0 - _ALPHA) / 3.0, 0.0)
    z1n = z1 * (1.0 / jnp.maximum(
        jnp.sqrt(jnp.sum(z1 * z1, axis=-1, keepdims=True)), 1e-12))
    z2n = z2 * (1.0 / jnp.maximum(
        jnp.sqrt(jnp.sum(z2 * z2, axis=-1, keepdims=True)), 1e-12))

    dot = functools.partial(jnp.dot, preferred_element_type=jnp.float32)
    s11 = dot(z1n, z1n.T)
    s22 = dot(z2n, z2n.T)
    s12 = dot(z1n, z2n.T)
    s21 = dot(z2n, z1n.T)

    a1 = _softmax_noshift(s11)
    a1_2 = _dot_bf16(a1, a1)
    asum1 = a1 + a1_2 + _dot_bf16(a1_2, a1_2)
    a2v = _softmax_noshift(s22)
    a2_2 = _dot_bf16(a2v, a2v)
    asum2 = a2v + a2_2 + _dot_bf16(a2_2, a2_2)

    inv_t = 1.0 / _TEMPERATURE
    e11 = jnp.exp(s11 * inv_t)
    e22 = jnp.exp(s22 * inv_t)
    e12 = jnp.exp(s12 * inv_t)
    e21 = jnp.exp(s21 * inv_t)

    diag_e11 = jnp.exp(jnp.sum(z1n * z1n, axis=-1, keepdims=True) * inv_t)
    diag_e22 = jnp.exp(jnp.sum(z2n * z2n, axis=-1, keepdims=True) * inv_t)
    strong = jnp.exp(jnp.sum(z1n * z2n, axis=-1, keepdims=True) * inv_t)

    den1 = (jnp.sum(e11, axis=-1, keepdims=True) - diag_e11
            + jnp.sum(e12, axis=-1, keepdims=True))
    den2 = (jnp.sum(e22, axis=-1, keepdims=True) - diag_e22
            + jnp.sum(e21, axis=-1, keepdims=True))

    num1 = strong + _band_num(asum2, e12, c1, c2)
    num2 = strong + _band_num(asum1, e21, c1, c2)

    li1 = -jnp.log(num1 / (den1 + _EPS) + _EPS)
    li2 = -jnp.log(num2 / (den2 + _EPS) + _EPS)
    out_ref[...] = (jnp.sum(li1) + jnp.sum(li2)).reshape(1, 1, 1)


def kernel(proj_z1, proj_z2):
    partial = pl.pallas_call(
        _loss_kernel,
        grid=(_B,),
        in_specs=[
            pl.BlockSpec((1, _N, _D), lambda b: (b, 0, 0)),
            pl.BlockSpec((1, _N, _D), lambda b: (b, 0, 0)),
        ],
        out_specs=pl.BlockSpec((1, 1, 1), lambda b: (b, 0, 0)),
        out_shape=jax.ShapeDtypeStruct((_B, 1, 1), jnp.float32),
    )(proj_z1, proj_z2)
    return _LOSS_W * jnp.sum(partial) / (_B * 2 * _N)


# speedup vs baseline: 1.1103x; 1.1103x over previous
import functools

import jax
import jax.numpy as jnp
from jax.experimental import pallas as pl

_B = 8
_N = 512
_D = 128
_TEMPERATURE = 0.07
_ALPHA = 0.5
_TOP_K = 5
_SIGMA = 2.0
_LOSS_W = 1.0
_EPS = 1e-09


def _softmax_noshift(s):
    e = jnp.exp(s)
    return e * (1.0 / jnp.sum(e, axis=-1, keepdims=True))


def _loss_kernel(z1_ref, z2_ref, out_ref):
    z1 = z1_ref[0]
    z2 = z2_ref[0]
    z1n = z1 * (1.0 / jnp.maximum(
        jnp.sqrt(jnp.sum(z1 * z1, axis=-1, keepdims=True)), 1e-12))
    z2n = z2 * (1.0 / jnp.maximum(
        jnp.sqrt(jnp.sum(z2 * z2, axis=-1, keepdims=True)), 1e-12))

    dot = functools.partial(jnp.dot, preferred_element_type=jnp.float32)
    s11 = dot(z1n, z1n.T)
    s22 = dot(z2n, z2n.T)
    s12 = dot(z1n, z2n.T)
    s21 = dot(z2n, z1n.T)

    def dot_bf16(x, y):
        return jnp.dot(x.astype(jnp.bfloat16), y.astype(jnp.bfloat16),
                       preferred_element_type=jnp.float32)

    a1 = _softmax_noshift(s11)
    a1_2 = dot_bf16(a1, a1)
    assoc1 = (a1 + a1_2 + dot_bf16(a1_2, a1_2)) * (1.0 / 3.0)
    a2 = _softmax_noshift(s22)
    a2_2 = dot_bf16(a2, a2)
    assoc2 = (a2 + a2_2 + dot_bf16(a2_2, a2_2)) * (1.0 / 3.0)

    inv_t = 1.0 / _TEMPERATURE
    e11 = jnp.exp(s11 * inv_t)
    e22 = jnp.exp(s22 * inv_t)
    e12 = jnp.exp(s12 * inv_t)
    e21 = jnp.exp(s21 * inv_t)

    diag_e11 = jnp.exp(jnp.sum(z1n * z1n, axis=-1, keepdims=True) * inv_t)
    diag_e22 = jnp.exp(jnp.sum(z2n * z2n, axis=-1, keepdims=True) * inv_t)
    strong = jnp.exp(jnp.sum(z1n * z2n, axis=-1, keepdims=True) * inv_t)

    den1 = (jnp.sum(e11, axis=-1, keepdims=True) - diag_e11
            + jnp.sum(e12, axis=-1, keepdims=True))
    den2 = (jnp.sum(e22, axis=-1, keepdims=True) - diag_e22
            + jnp.sum(e21, axis=-1, keepdims=True))

    n = _N
    col = jax.lax.broadcasted_iota(jnp.int32, (n, n), 1)
    row = jax.lax.broadcasted_iota(jnp.int32, (n, n), 0)

    start = jnp.clip(row - 2, 0, n - _TOP_K)
    w = (col >= start) & (col < start + _TOP_K) & (col != row)

    d = (row - col).astype(jnp.float32)
    pos_w = _ALPHA * jnp.exp(-(d * d) * (1.0 / (2.0 * _SIGMA * _SIGMA)))
    score1 = pos_w + (1.0 - _ALPHA) * assoc1
    score2 = pos_w + (1.0 - _ALPHA) * assoc2

    num1 = strong + jnp.sum(
        jnp.where(w, score2 * e12, 0.0), axis=-1, keepdims=True)
    num2 = strong + jnp.sum(
        jnp.where(w, score1 * e21, 0.0), axis=-1, keepdims=True)

    li1 = -jnp.log(num1 / (den1 + _EPS) + _EPS)
    li2 = -jnp.log(num2 / (den2 + _EPS) + _EPS)
    out_ref[...] = (jnp.sum(li1) + jnp.sum(li2)).reshape(1, 1, 1)


def kernel(proj_z1, proj_z2):
    partial = pl.pallas_call(
        _loss_kernel,
        grid=(_B,),
        in_specs=[
            pl.BlockSpec((1, _N, _D), lambda b: (b, 0, 0)),
            pl.BlockSpec((1, _N, _D), lambda b: (b, 0, 0)),
        ],
        out_specs=pl.BlockSpec((1, 1, 1), lambda b: (b, 0, 0)),
        out_shape=jax.ShapeDtypeStruct((_B, 1, 1), jnp.float32),
    )(proj_z1, proj_z2)
    return _LOSS_W * jnp.sum(partial) / (_B * 2 * _N)


# speedup vs baseline: 1.1210x; 1.0097x over previous
import functools

import jax
import jax.numpy as jnp
from jax.experimental import pallas as pl

_B = 8
_N = 512
_D = 128
_TEMPERATURE = 0.07
_ALPHA = 0.5
_TOP_K = 5
_SIGMA = 2.0
_LOSS_W = 1.0
_EPS = 1e-09


def _softmax_noshift(s):
    e = jnp.exp(s)
    return e * (1.0 / jnp.sum(e, axis=-1, keepdims=True))


def _sample_loss(z1, z2):
    z1n = z1 * (1.0 / jnp.maximum(
        jnp.sqrt(jnp.sum(z1 * z1, axis=-1, keepdims=True)), 1e-12))
    z2n = z2 * (1.0 / jnp.maximum(
        jnp.sqrt(jnp.sum(z2 * z2, axis=-1, keepdims=True)), 1e-12))

    dot = functools.partial(jnp.dot, preferred_element_type=jnp.float32)
    s11 = dot(z1n, z1n.T)
    s22 = dot(z2n, z2n.T)
    s12 = dot(z1n, z2n.T)
    s21 = dot(z2n, z1n.T)

    def dot_bf16(x, y):
        return jnp.dot(x.astype(jnp.bfloat16), y.astype(jnp.bfloat16),
                       preferred_element_type=jnp.float32)

    a1 = _softmax_noshift(s11)
    a1_2 = dot_bf16(a1, a1)
    assoc1 = (a1 + a1_2 + dot_bf16(a1_2, a1_2)) * (1.0 / 3.0)
    a2 = _softmax_noshift(s22)
    a2_2 = dot_bf16(a2, a2)
    assoc2 = (a2 + a2_2 + dot_bf16(a2_2, a2_2)) * (1.0 / 3.0)

    inv_t = 1.0 / _TEMPERATURE
    e11 = jnp.exp(s11 * inv_t)
    e22 = jnp.exp(s22 * inv_t)
    e12 = jnp.exp(s12 * inv_t)
    e21 = jnp.exp(s21 * inv_t)

    diag_e11 = jnp.exp(jnp.sum(z1n * z1n, axis=-1, keepdims=True) * inv_t)
    diag_e22 = jnp.exp(jnp.sum(z2n * z2n, axis=-1, keepdims=True) * inv_t)
    strong = jnp.exp(jnp.sum(z1n * z2n, axis=-1, keepdims=True) * inv_t)

    den1 = (jnp.sum(e11, axis=-1, keepdims=True) - diag_e11
            + jnp.sum(e12, axis=-1, keepdims=True))
    den2 = (jnp.sum(e22, axis=-1, keepdims=True) - diag_e22
            + jnp.sum(e21, axis=-1, keepdims=True))

    n = _N
    col = jax.lax.broadcasted_iota(jnp.int32, (n, n), 1)
    row = jax.lax.broadcasted_iota(jnp.int32, (n, n), 0)

    start = jnp.clip(row - 2, 0, n - _TOP_K)
    w = (col >= start) & (col < start + _TOP_K) & (col != row)

    d = (row - col).astype(jnp.float32)
    pos_w = _ALPHA * jnp.exp(-(d * d) * (1.0 / (2.0 * _SIGMA * _SIGMA)))
    score1 = pos_w + (1.0 - _ALPHA) * assoc1
    score2 = pos_w + (1.0 - _ALPHA) * assoc2

    num1 = strong + jnp.sum(
        jnp.where(w, score2 * e12, 0.0), axis=-1, keepdims=True)
    num2 = strong + jnp.sum(
        jnp.where(w, score1 * e21, 0.0), axis=-1, keepdims=True)

    li1 = -jnp.log(num1 / (den1 + _EPS) + _EPS)
    li2 = -jnp.log(num2 / (den2 + _EPS) + _EPS)
    return jnp.sum(li1) + jnp.sum(li2)


_SPP = 2


def _loss_kernel(z1_ref, z2_ref, out_ref):
    total = _sample_loss(z1_ref[0], z2_ref[0])
    for s in range(1, _SPP):
        total = total + _sample_loss(z1_ref[s], z2_ref[s])
    out_ref[...] = total.reshape(1, 1, 1)


def kernel(proj_z1, proj_z2):
    partial = pl.pallas_call(
        _loss_kernel,
        grid=(_B // _SPP,),
        in_specs=[
            pl.BlockSpec((_SPP, _N, _D), lambda b: (b, 0, 0)),
            pl.BlockSpec((_SPP, _N, _D), lambda b: (b, 0, 0)),
        ],
        out_specs=pl.BlockSpec((1, 1, 1), lambda b: (b, 0, 0)),
        out_shape=jax.ShapeDtypeStruct((_B // _SPP, 1, 1), jnp.float32),
    )(proj_z1, proj_z2)
    return _LOSS_W * jnp.sum(partial) / (_B * 2 * _N)


# speedup vs baseline: 1.1444x; 1.0209x over previous
import functools

import jax
import jax.numpy as jnp
from jax.experimental import pallas as pl

_B = 8
_N = 512
_D = 128
_TEMPERATURE = 0.07
_ALPHA = 0.5
_TOP_K = 5
_SIGMA = 2.0
_LOSS_W = 1.0
_EPS = 1e-09


def _softmax_noshift(s):
    e = jnp.exp(s)
    return e * (1.0 / jnp.sum(e, axis=-1, keepdims=True))


def _sample_loss(z1, z2):
    z1n = z1 * (1.0 / jnp.maximum(
        jnp.sqrt(jnp.sum(z1 * z1, axis=-1, keepdims=True)), 1e-12))
    z2n = z2 * (1.0 / jnp.maximum(
        jnp.sqrt(jnp.sum(z2 * z2, axis=-1, keepdims=True)), 1e-12))

    dot = functools.partial(jnp.dot, preferred_element_type=jnp.float32)
    s11 = dot(z1n, z1n.T)
    s22 = dot(z2n, z2n.T)
    s12 = dot(z1n, z2n.T)
    s21 = dot(z2n, z1n.T)

    def dot_bf16(x, y):
        return jnp.dot(x.astype(jnp.bfloat16), y.astype(jnp.bfloat16),
                       preferred_element_type=jnp.float32)

    a1 = _softmax_noshift(s11)
    a1_2 = dot_bf16(a1, a1)
    assoc1 = (a1 + a1_2 + dot_bf16(a1_2, a1_2)) * (1.0 / 3.0)
    a2 = _softmax_noshift(s22)
    a2_2 = dot_bf16(a2, a2)
    assoc2 = (a2 + a2_2 + dot_bf16(a2_2, a2_2)) * (1.0 / 3.0)

    inv_t = 1.0 / _TEMPERATURE
    e11 = jnp.exp(s11 * inv_t)
    e22 = jnp.exp(s22 * inv_t)
    e12 = jnp.exp(s12 * inv_t)
    e21 = jnp.exp(s21 * inv_t)

    diag_e11 = jnp.exp(jnp.sum(z1n * z1n, axis=-1, keepdims=True) * inv_t)
    diag_e22 = jnp.exp(jnp.sum(z2n * z2n, axis=-1, keepdims=True) * inv_t)
    strong = jnp.exp(jnp.sum(z1n * z2n, axis=-1, keepdims=True) * inv_t)

    den1 = (jnp.sum(e11, axis=-1, keepdims=True) - diag_e11
            + jnp.sum(e12, axis=-1, keepdims=True))
    den2 = (jnp.sum(e22, axis=-1, keepdims=True) - diag_e22
            + jnp.sum(e21, axis=-1, keepdims=True))

    n = _N
    col = jax.lax.broadcasted_iota(jnp.int32, (n, n), 1)
    row = jax.lax.broadcasted_iota(jnp.int32, (n, n), 0)

    start = jnp.clip(row - 2, 0, n - _TOP_K)
    w = (col >= start) & (col < start + _TOP_K) & (col != row)

    d = (row - col).astype(jnp.float32)
    pos_w = _ALPHA * jnp.exp(-(d * d) * (1.0 / (2.0 * _SIGMA * _SIGMA)))
    score1 = pos_w + (1.0 - _ALPHA) * assoc1
    score2 = pos_w + (1.0 - _ALPHA) * assoc2

    num1 = strong + jnp.sum(
        jnp.where(w, score2 * e12, 0.0), axis=-1, keepdims=True)
    num2 = strong + jnp.sum(
        jnp.where(w, score1 * e21, 0.0), axis=-1, keepdims=True)

    li1 = -jnp.log(num1 / (den1 + _EPS) + _EPS)
    li2 = -jnp.log(num2 / (den2 + _EPS) + _EPS)
    return jnp.sum(li1) + jnp.sum(li2)


_SPP = 4


def _loss_kernel(z1_ref, z2_ref, out_ref):
    total = _sample_loss(z1_ref[0], z2_ref[0])
    for s in range(1, _SPP):
        total = total + _sample_loss(z1_ref[s], z2_ref[s])
    out_ref[...] = total.reshape(1, 1, 1)


def kernel(proj_z1, proj_z2):
    partial = pl.pallas_call(
        _loss_kernel,
        grid=(_B // _SPP,),
        in_specs=[
            pl.BlockSpec((_SPP, _N, _D), lambda b: (b, 0, 0)),
            pl.BlockSpec((_SPP, _N, _D), lambda b: (b, 0, 0)),
        ],
        out_specs=pl.BlockSpec((1, 1, 1), lambda b: (b, 0, 0)),
        out_shape=jax.ShapeDtypeStruct((_B // _SPP, 1, 1), jnp.float32),
    )(proj_z1, proj_z2)
    return _LOSS_W * jnp.sum(partial) / (_B * 2 * _N)


# speedup vs baseline: 1.1847x; 1.0352x over previous
import functools

import jax
import jax.numpy as jnp
from jax.experimental import pallas as pl

_B = 8
_N = 512
_D = 128
_TEMPERATURE = 0.07
_ALPHA = 0.5
_TOP_K = 5
_SIGMA = 2.0
_LOSS_W = 1.0
_EPS = 1e-09


def _softmax_noshift(s):
    e = jnp.exp(s)
    return e * (1.0 / jnp.sum(e, axis=-1, keepdims=True))


def _sample_loss(z1, z2):
    z1n = z1 * (1.0 / jnp.maximum(
        jnp.sqrt(jnp.sum(z1 * z1, axis=-1, keepdims=True)), 1e-12))
    z2n = z2 * (1.0 / jnp.maximum(
        jnp.sqrt(jnp.sum(z2 * z2, axis=-1, keepdims=True)), 1e-12))

    dot = functools.partial(jnp.dot, preferred_element_type=jnp.float32)
    s11 = dot(z1n, z1n.T)
    s22 = dot(z2n, z2n.T)
    s12 = dot(z1n, z2n.T)
    s21 = dot(z2n, z1n.T)

    def dot_bf16(x, y):
        return jnp.dot(x.astype(jnp.bfloat16), y.astype(jnp.bfloat16),
                       preferred_element_type=jnp.float32)

    a1 = _softmax_noshift(s11)
    a1_2 = dot_bf16(a1, a1)
    assoc1 = (a1 + a1_2 + dot_bf16(a1_2, a1_2)) * (1.0 / 3.0)
    a2 = _softmax_noshift(s22)
    a2_2 = dot_bf16(a2, a2)
    assoc2 = (a2 + a2_2 + dot_bf16(a2_2, a2_2)) * (1.0 / 3.0)

    inv_t = 1.0 / _TEMPERATURE
    e11 = jnp.exp(s11 * inv_t)
    e22 = jnp.exp(s22 * inv_t)
    e12 = jnp.exp(s12 * inv_t)
    e21 = jnp.exp(s21 * inv_t)

    diag_e11 = jnp.exp(jnp.sum(z1n * z1n, axis=-1, keepdims=True) * inv_t)
    diag_e22 = jnp.exp(jnp.sum(z2n * z2n, axis=-1, keepdims=True) * inv_t)
    strong = jnp.exp(jnp.sum(z1n * z2n, axis=-1, keepdims=True) * inv_t)

    den1 = (jnp.sum(e11, axis=-1, keepdims=True) - diag_e11
            + jnp.sum(e12, axis=-1, keepdims=True))
    den2 = (jnp.sum(e22, axis=-1, keepdims=True) - diag_e22
            + jnp.sum(e21, axis=-1, keepdims=True))

    n = _N
    col = jax.lax.broadcasted_iota(jnp.int32, (n, n), 1)
    row = jax.lax.broadcasted_iota(jnp.int32, (n, n), 0)

    start = jnp.clip(row - 2, 0, n - _TOP_K)
    w = (col >= start) & (col < start + _TOP_K) & (col != row)

    d = (row - col).astype(jnp.float32)
    pos_w = _ALPHA * jnp.exp(-(d * d) * (1.0 / (2.0 * _SIGMA * _SIGMA)))
    score1 = pos_w + (1.0 - _ALPHA) * assoc1
    score2 = pos_w + (1.0 - _ALPHA) * assoc2

    num1 = strong + jnp.sum(
        jnp.where(w, score2 * e12, 0.0), axis=-1, keepdims=True)
    num2 = strong + jnp.sum(
        jnp.where(w, score1 * e21, 0.0), axis=-1, keepdims=True)

    li1 = -jnp.log(num1 / (den1 + _EPS) + _EPS)
    li2 = -jnp.log(num2 / (den2 + _EPS) + _EPS)
    return jnp.sum(li1) + jnp.sum(li2)


_SPP = 8


def _loss_kernel(z1_ref, z2_ref, out_ref):
    total = _sample_loss(z1_ref[0], z2_ref[0])
    for s in range(1, _SPP):
        total = total + _sample_loss(z1_ref[s], z2_ref[s])
    out_ref[...] = total.reshape(1, 1, 1)


def kernel(proj_z1, proj_z2):
    partial = pl.pallas_call(
        _loss_kernel,
        grid=(_B // _SPP,),
        in_specs=[
            pl.BlockSpec((_SPP, _N, _D), lambda b: (b, 0, 0)),
            pl.BlockSpec((_SPP, _N, _D), lambda b: (b, 0, 0)),
        ],
        out_specs=pl.BlockSpec((1, 1, 1), lambda b: (b, 0, 0)),
        out_shape=jax.ShapeDtypeStruct((_B // _SPP, 1, 1), jnp.float32),
    )(proj_z1, proj_z2)
    return _LOSS_W * jnp.sum(partial) / (_B * 2 * _N)


# speedup vs baseline: 1.1864x; 1.0014x over previous
import functools

import jax
import jax.numpy as jnp
from jax.experimental import pallas as pl

_B = 8
_N = 512
_D = 128
_TEMPERATURE = 0.07
_ALPHA = 0.5
_TOP_K = 5
_SIGMA = 2.0
_LOSS_W = 1.0
_EPS = 1e-09


def _softmax_noshift(s):
    e = jnp.exp(s)
    return e * (1.0 / jnp.sum(e, axis=-1, keepdims=True))


def _sample_loss(z1, z2):
    z1n = z1 * (1.0 / jnp.maximum(
        jnp.sqrt(jnp.sum(z1 * z1, axis=-1, keepdims=True)), 1e-12))
    z2n = z2 * (1.0 / jnp.maximum(
        jnp.sqrt(jnp.sum(z2 * z2, axis=-1, keepdims=True)), 1e-12))

    zc = jnp.concatenate([z1n, z2n], axis=0).astype(jnp.bfloat16)
    s = jnp.dot(zc, zc.T, preferred_element_type=jnp.float32)
    s11 = s[:_N, :_N]
    s22 = s[_N:, _N:]

    def dot_bf16(x, y):
        return jnp.dot(x.astype(jnp.bfloat16), y.astype(jnp.bfloat16),
                       preferred_element_type=jnp.float32)

    a1 = _softmax_noshift(s11)
    a1_2 = dot_bf16(a1, a1)
    assoc1 = (a1 + a1_2 + dot_bf16(a1_2, a1_2)) * (1.0 / 3.0)
    a2 = _softmax_noshift(s22)
    a2_2 = dot_bf16(a2, a2)
    assoc2 = (a2 + a2_2 + dot_bf16(a2_2, a2_2)) * (1.0 / 3.0)

    inv_t = 1.0 / _TEMPERATURE
    e = jnp.exp(s * inv_t)
    e12 = e[:_N, _N:]
    e21 = e[_N:, :_N]

    zcf = zc.astype(jnp.float32)
    diag_q = jnp.exp(jnp.sum(zcf * zcf, axis=-1, keepdims=True) * inv_t)
    strong = jnp.exp(
        jnp.sum(zcf[:_N] * zcf[_N:], axis=-1, keepdims=True) * inv_t)

    den = jnp.sum(e, axis=-1, keepdims=True) - diag_q
    den1 = den[:_N]
    den2 = den[_N:]

    n = _N
    col = jax.lax.broadcasted_iota(jnp.int32, (n, n), 1)
    row = jax.lax.broadcasted_iota(jnp.int32, (n, n), 0)

    start = jnp.clip(row - 2, 0, n - _TOP_K)
    w = (col >= start) & (col < start + _TOP_K) & (col != row)

    d = (row - col).astype(jnp.float32)
    pos_w = _ALPHA * jnp.exp(-(d * d) * (1.0 / (2.0 * _SIGMA * _SIGMA)))
    score1 = pos_w + (1.0 - _ALPHA) * assoc1
    score2 = pos_w + (1.0 - _ALPHA) * assoc2

    num1 = strong + jnp.sum(
        jnp.where(w, score2 * e12, 0.0), axis=-1, keepdims=True)
    num2 = strong + jnp.sum(
        jnp.where(w, score1 * e21, 0.0), axis=-1, keepdims=True)

    li1 = -jnp.log(num1 / (den1 + _EPS) + _EPS)
    li2 = -jnp.log(num2 / (den2 + _EPS) + _EPS)
    return jnp.sum(li1) + jnp.sum(li2)


_SPP = 8


def _loss_kernel(z1_ref, z2_ref, out_ref):
    total = _sample_loss(z1_ref[0], z2_ref[0])
    for s in range(1, _SPP):
        total = total + _sample_loss(z1_ref[s], z2_ref[s])
    out_ref[...] = total.reshape(1, 1, 1)


def kernel(proj_z1, proj_z2):
    partial = pl.pallas_call(
        _loss_kernel,
        grid=(_B // _SPP,),
        in_specs=[
            pl.BlockSpec((_SPP, _N, _D), lambda b: (b, 0, 0)),
            pl.BlockSpec((_SPP, _N, _D), lambda b: (b, 0, 0)),
        ],
        out_specs=pl.BlockSpec((1, 1, 1), lambda b: (b, 0, 0)),
        out_shape=jax.ShapeDtypeStruct((_B // _SPP, 1, 1), jnp.float32),
    )(proj_z1, proj_z2)
    return _LOSS_W * jnp.sum(partial) / (_B * 2 * _N)


# speedup vs baseline: 1.1888x; 1.0020x over previous
import jax
import jax.numpy as jnp
from jax.experimental import pallas as pl

_B = 8
_N = 512
_D = 128
_TEMPERATURE = 0.07
_ALPHA = 0.5
_TOP_K = 5
_SIGMA = 2.0
_LOSS_W = 1.0
_EPS = 1e-09


def _softmax_noshift(s):
    e = jnp.exp(s)
    return e * (1.0 / jnp.sum(e, axis=-1, keepdims=True))


def _sample_loss(z1, z2):
    z1n = z1 * (1.0 / jnp.maximum(
        jnp.sqrt(jnp.sum(z1 * z1, axis=-1, keepdims=True)), 1e-12))
    z2n = z2 * (1.0 / jnp.maximum(
        jnp.sqrt(jnp.sum(z2 * z2, axis=-1, keepdims=True)), 1e-12))

    zc = jnp.concatenate([z1n, z2n], axis=0).astype(jnp.bfloat16)
    s = jnp.dot(zc, zc.T, preferred_element_type=jnp.float32)
    s11 = s[:_N, :_N]
    s22 = s[_N:, _N:]

    def dot_bf16(x, y):
        return jnp.dot(x.astype(jnp.bfloat16), y.astype(jnp.bfloat16),
                       preferred_element_type=jnp.float32)

    a1 = _softmax_noshift(s11)
    a1_2 = dot_bf16(a1, a1)
    assoc1 = (a1 + a1_2 + dot_bf16(a1_2, a1_2)) * (1.0 / 3.0)
    a2 = _softmax_noshift(s22)
    a2_2 = dot_bf16(a2, a2)
    assoc2 = (a2 + a2_2 + dot_bf16(a2_2, a2_2)) * (1.0 / 3.0)

    inv_t = 1.0 / _TEMPERATURE
    e = jnp.exp(s * inv_t)
    e12 = e[:_N, _N:]
    e21 = e[_N:, :_N]

    zcf = zc.astype(jnp.float32)
    diag_q = jnp.exp(jnp.sum(zcf * zcf, axis=-1, keepdims=True) * inv_t)
    strong = jnp.exp(
        jnp.sum(zcf[:_N] * zcf[_N:], axis=-1, keepdims=True) * inv_t)

    den = jnp.sum(e, axis=-1, keepdims=True) - diag_q
    den1 = den[:_N]
    den2 = den[_N:]

    n = _N
    col = jax.lax.broadcasted_iota(jnp.int32, (n, n), 1)
    row = jax.lax.broadcasted_iota(jnp.int32, (n, n), 0)

    start = jnp.clip(row - 2, 0, n - _TOP_K)
    w = (col >= start) & (col < start + _TOP_K) & (col != row)

    d = (row - col).astype(jnp.float32)
    pos_w = _ALPHA * jnp.exp(-(d * d) * (1.0 / (2.0 * _SIGMA * _SIGMA)))
    score1 = pos_w + (1.0 - _ALPHA) * assoc1
    score2 = pos_w + (1.0 - _ALPHA) * assoc2

    num1 = strong + jnp.sum(
        jnp.where(w, score2 * e12, 0.0), axis=-1, keepdims=True)
    num2 = strong + jnp.sum(
        jnp.where(w, score1 * e21, 0.0), axis=-1, keepdims=True)

    li1 = -jnp.log(num1 / (den1 + _EPS) + _EPS)
    li2 = -jnp.log(num2 / (den2 + _EPS) + _EPS)
    return jnp.sum(li1) + jnp.sum(li2)


_SPP = 8


def _loss_kernel(z1_ref, z2_ref, out_ref):
    total = _sample_loss(z1_ref[0], z2_ref[0])
    for s in range(1, _SPP):
        total = total + _sample_loss(z1_ref[s], z2_ref[s])
    out_ref[...] = total.reshape(1, 1, 1)


def kernel(proj_z1, proj_z2):
    partial = pl.pallas_call(
        _loss_kernel,
        grid=(_B // _SPP,),
        in_specs=[
            pl.BlockSpec((_SPP, _N, _D), lambda b: (b, 0, 0)),
            pl.BlockSpec((_SPP, _N, _D), lambda b: (b, 0, 0)),
        ],
        out_specs=pl.BlockSpec((1, 1, 1), lambda b: (b, 0, 0)),
        out_shape=jax.ShapeDtypeStruct((_B // _SPP, 1, 1), jnp.float32),
    )(proj_z1, proj_z2)
    return _LOSS_W * jnp.sum(partial) / (_B * 2 * _N)
